# baseline (speedup 1.0000x reference)
import functools

import jax
import jax.numpy as jnp
from jax import lax
from jax.experimental import pallas as pl
from jax.experimental.pallas import tpu as pltpu
from jax.experimental.pallas import tpu_sc as plsc

NN = 10000
EE = 320000
H = 8
C = 8
DH = 64

NC = 2
NS = 16
LANES = 16

EDGES_PER_TILE = EE // (NC * NS)
BLK = 80
NBLK = EDGES_PER_TILE // BLK
NPAD = 10240
ROWS_PER_TILE = NPAD // NS
ZCH = 80

RB = 1000
NRB = NN // RB


NBUF = 3


def _make_edge_kernel(width, with_ones):
  mesh = plsc.VectorSubcoreMesh(core_axis_name="c", subcore_axis_name="s")

  @functools.partial(
      pl.kernel,
      out_type=jax.ShapeDtypeStruct((NC, NPAD, width), jnp.float32),
      mesh=mesh,
      scratch_types=[
          pltpu.VMEM_SHARED((NPAD, width), jnp.float32),
          pltpu.VMEM((NBLK, BLK), jnp.int32),
          pltpu.VMEM((NBLK, BLK), jnp.int32),
          [pltpu.VMEM((BLK, DH), jnp.float32) for _ in range(NBUF)],
          [pltpu.VMEM((BLK, DH), jnp.float32) for _ in range(NBUF)],
          [pltpu.VMEM((BLK, width), jnp.float32) for _ in range(NBUF)],
          [pltpu.VMEM((BLK,), jnp.int32) for _ in range(NBUF)],
          pltpu.VMEM((80,), jnp.float32),
          [pltpu.SemaphoreType.DMA for _ in range(NBUF)],
          [pltpu.SemaphoreType.DMA for _ in range(NBUF)],
      ],
      compiler_params=pltpu.CompilerParams(
          needs_layout_passes=False, use_tc_tiling_on_sc=False),
  )
  def edge_kernel(xl_hbm, xr_hbm, src_hbm, dst_hbm, att_hbm, z_hbm,
                  acc_out, acc_sh, src_v, dst_v, xl_v, xr_v, prod_v,
                  sidx_v, att_v, gsem, ssem):
    cid = lax.axis_index("c")
    sid = lax.axis_index("s")
    tid = cid * NS + sid

    pltpu.sync_copy(att_hbm, att_v.at[pl.ds(16, 64)])
    pltpu.sync_copy(src_hbm.at[tid], src_v)
    pltpu.sync_copy(dst_hbm.at[tid], dst_v)

    for j in range(ROWS_PER_TILE // ZCH):
      off = sid * ROWS_PER_TILE + j * ZCH
      pltpu.sync_copy(z_hbm, acc_sh.at[pl.ds(off, ZCH)])

    if with_ones:
      ones16 = jnp.ones((LANES,), jnp.float32)
      for s in range(NBUF):
        for g in range(BLK // LANES):
          rows = lax.iota(jnp.int32, LANES) + g * LANES
          for cc in range(DH + H, width):
            plsc.store_scatter(prod_v[s],
                               [rows, jnp.full((LANES,), cc, jnp.int32)],
                               ones16)

    plsc.subcore_barrier()

    def issue_gather(b, s):
      pltpu.async_copy(xl_hbm.at[src_v.at[b]], xl_v[s], gsem[s])
      pltpu.async_copy(xr_hbm.at[dst_v.at[b]], xr_v[s], gsem[s])

    def compute(b, s):
      @plsc.parallel_loop(0, (BLK // LANES) * H, unroll=4)
      def _(i, s=s):
        h = lax.bitwise_and(i, H - 1)
        g = lax.shift_right_logical(i, 3)
        rows = lax.iota(jnp.int32, LANES) + g * LANES
        colbase = h * C
        acc = jnp.zeros((LANES,), jnp.float32)
        xls = []
        for c in range(C):
          col = jnp.full((LANES,), c, jnp.int32) + colbase
          attv = plsc.load_gather(att_v, [col + 16])
          xlv = plsc.load_gather(xl_v[s], [rows, col])
          xrv = plsc.load_gather(xr_v[s], [rows, col])
          sv = xlv + xrv
          sv = jnp.maximum(sv, 0.2 * sv)
          acc = acc + sv * attv
          xls.append((col, xlv))
        exh = jnp.exp(acc)
        plsc.store_scatter(prod_v[s],
                           [rows, jnp.full((LANES,), DH, jnp.int32) + h],
                           exh)
        for col, xlv in xls:
          plsc.store_scatter(prod_v[s], [rows, col], exh * xlv)
      for g in range(BLK // LANES):
        sidx_v[s][pl.ds(g * LANES, LANES)] = dst_v[b, pl.ds(g * LANES, LANES)]

    issue_gather(0, 0)
    issue_gather(1, 1)

    def step(b, s):
      sp = (s + 2) % NBUF

      @pl.when(b + 2 < NBLK)
      def _(b=b, sp=sp):
        issue_gather(b + 2, sp)

      pltpu.make_async_copy(xl_hbm.at[src_v.at[b]], xl_v[s], gsem[s]).wait()
      pltpu.make_async_copy(xr_hbm.at[dst_v.at[b]], xr_v[s], gsem[s]).wait()

      @pl.when(b >= NBUF)
      def _(s=s):
        pltpu.make_async_copy(prod_v[s], acc_sh.at[sidx_v[s]], ssem[s]).wait()

      compute(b, s)
      pltpu.async_copy(prod_v[s], acc_sh.at[sidx_v[s]], ssem[s], add=True)

    nfull = NBLK // NBUF

    def outer(o, carry):
      for s in range(NBUF):
        step(o * NBUF + s, s)
      return carry

    lax.fori_loop(0, nfull, outer, 0)
    for b in range(nfull * NBUF, NBLK):
      step(jnp.int32(b), b % NBUF)
    for s in range(NBUF):
      pltpu.make_async_copy(prod_v[s], acc_sh.at[sidx_v[s]], ssem[s]).wait()

    plsc.subcore_barrier()

    for j in range(ROWS_PER_TILE // ZCH):
      off = sid * ROWS_PER_TILE + j * ZCH
      pltpu.sync_copy(acc_sh.at[pl.ds(off, ZCH)],
                      acc_out.at[cid, pl.ds(off, ZCH)])

  return edge_kernel


_edge_sum = _make_edge_kernel(DH + H, False)
_edge_mean = _make_edge_kernel(DH + H + H, True)


def _elu(t):
  return jnp.where(t > 0, t, jnp.exp(jnp.minimum(t, 0.0)) - 1.0)


def _bn_full(t, g, b):
  m = jnp.mean(t, axis=0, keepdims=True)
  v = jnp.mean((t - m) * (t - m), axis=0, keepdims=True)
  return (t - m) / jnp.sqrt(v + 1e-5) * g + b


def _pre_body(x_ref, w1_ref, g1_ref, b1_ref, w2_ref, g2_ref, b2_ref, h_ref):
  t = jnp.dot(x_ref[...], w1_ref[...], preferred_element_type=jnp.float32)
  t = _elu(_bn_full(t, g1_ref[...], b1_ref[...]))
  t = jnp.dot(t, w2_ref[...], preferred_element_type=jnp.float32)
  h_ref[...] = _elu(_bn_full(t, g2_ref[...], b2_ref[...]))


def _pre(x, w1, g1, b1, w2, g2, b2):
  return pl.pallas_call(
      _pre_body,
      out_shape=jax.ShapeDtypeStruct((NN, DH), jnp.float32),
  )(x, w1, g1, b1, w2, g2, b2)


def _mm4_body(h_ref, wa_ref, wb_ref, wc_ref, wd_ref, oa, ob, oc, od):
  hv = h_ref[...]
  oa[...] = jnp.dot(hv, wa_ref[...], preferred_element_type=jnp.float32)
  ob[...] = jnp.dot(hv, wb_ref[...], preferred_element_type=jnp.float32)
  oc[...] = jnp.dot(hv, wc_ref[...], preferred_element_type=jnp.float32)
  od[...] = jnp.dot(hv, wd_ref[...], preferred_element_type=jnp.float32)


def _mm4(h, wa, wb, wc, wd):
  blk = pl.BlockSpec((RB, DH), lambda i: (i, 0))
  wspec = pl.BlockSpec((DH, DH), lambda i: (0, 0))
  return pl.pallas_call(
      _mm4_body,
      grid=(NRB,),
      in_specs=[blk, wspec, wspec, wspec, wspec],
      out_specs=[blk, blk, blk, blk],
      out_shape=[jax.ShapeDtypeStruct((NN, DH), jnp.float32)] * 4,
  )(h, wa, wb, wc, wd)


def _expand_mat():
  kk = lax.broadcasted_iota(jnp.int32, (H, DH), 1) // C
  hh = lax.broadcasted_iota(jnp.int32, (H, DH), 0)
  return (kk == hh).astype(jnp.float32)


def _b1_body(ndl_ref, ndr_ref, hl_ref, hr_ref, sl_ref, sr_ref, st_ref):
  i = pl.program_id(0)
  k = _expand_mat()
  ndl = ndl_ref[0] + ndl_ref[1]
  ndr = ndr_ref[0] + ndr_ref[1]
  den_l = jnp.dot(ndl[:, DH:DH + H], k, preferred_element_type=jnp.float32)
  gl = ndl[:, :DH] / (den_l + 1e-16)
  den_r = jnp.dot(ndr[:, DH:DH + H], k, preferred_element_type=jnp.float32)
  cnt = jnp.maximum(ndr[:, DH + H:DH + H + 1], 1.0)
  gr = ndr[:, :DH] / (den_r + 1e-16) / cnt
  sl = gl + hl_ref[...]
  sr = gr + hr_ref[...]
  sl_ref[...] = sl
  sr_ref[...] = sr
  z = jnp.zeros((1, DH), jnp.float32)
  st = jnp.concatenate(
      [jnp.sum(sl, axis=0, keepdims=True),
       jnp.sum(sl * sl, axis=0, keepdims=True),
       jnp.sum(sr, axis=0, keepdims=True),
       jnp.sum(sr * sr, axis=0, keepdims=True), z, z, z, z], axis=0)

  @pl.when(i == 0)
  def _():
    st_ref[...] = st

  @pl.when(i != 0)
  def _():
    st_ref[...] = st_ref[...] + st


def _b1(ndl, ndr, hl, hr):
  blk = pl.BlockSpec((RB, DH), lambda i: (i, 0))
  return pl.pallas_call(
      _b1_body,
      grid=(NRB,),
      in_specs=[
          pl.BlockSpec((NC, RB, DH + H), lambda i: (0, i, 0)),
          pl.BlockSpec((NC, RB, DH + 2 * H), lambda i: (0, i, 0)),
          blk, blk,
      ],
      out_specs=[blk, blk, pl.BlockSpec((8, DH), lambda i: (0, 0))],
      out_shape=[
          jax.ShapeDtypeStruct((NN, DH), jnp.float32),
          jax.ShapeDtypeStruct((NN, DH), jnp.float32),
          jax.ShapeDtypeStruct((8, DH), jnp.float32),
      ],
      compiler_params=pltpu.CompilerParams(
          dimension_semantics=("arbitrary",)),
  )(ndl, ndr, hl, hr)


def _bn_apply(s, st, row0, g, b):
  m = st[row0:row0 + 1] * (1.0 / NN)
  v = st[row0 + 1:row0 + 2] * (1.0 / NN) - m * m
  return (s - m) / jnp.sqrt(v + 1e-5) * g + b


def _b2_body(sl_ref, sr_ref, st_ref, lg_ref, lb_ref, rg_ref, rb_ref,
             wla_ref, wlb_ref, wra_ref, wrb_ref,
             hl_o, hr_o, xll_o, xrl_o, xlr_o, xrr_o):
  st = st_ref[...]
  hl = _elu(_bn_apply(sl_ref[...], st, 0, lg_ref[...], lb_ref[...]))
  hr = _elu(_bn_apply(sr_ref[...], st, 2, rg_ref[...], rb_ref[...]))
  hl_o[...] = hl
  hr_o[...] = hr
  xll_o[...] = jnp.dot(hl, wla_ref[...], preferred_element_type=jnp.float32)
  xrl_o[...] = jnp.dot(hl, wlb_ref[...], preferred_element_type=jnp.float32)
  xlr_o[...] = jnp.dot(hr, wra_ref[...], preferred_element_type=jnp.float32)
  xrr_o[...] = jnp.dot(hr, wrb_ref[...], preferred_element_type=jnp.float32)


def _b2(sl, sr, st, lg, lb, rg, rb, wla, wlb, wra, wrb):
  blk = pl.BlockSpec((RB, DH), lambda i: (i, 0))
  stspec = pl.BlockSpec((8, DH), lambda i: (0, 0))
  vec = pl.BlockSpec((1, DH), lambda i: (0, 0))
  wspec = pl.BlockSpec((DH, DH), lambda i: (0, 0))
  return pl.pallas_call(
      _b2_body,
      grid=(NRB,),
      in_specs=[blk, blk, stspec, vec, vec, vec, vec,
                wspec, wspec, wspec, wspec],
      out_specs=[blk] * 6,
      out_shape=[jax.ShapeDtypeStruct((NN, DH), jnp.float32)] * 6,
  )(sl, sr, st, lg, lb, rg, rb, wla, wlb, wra, wrb)


def _c2_body(sl_ref, sr_ref, st_ref, lg_ref, lb_ref, rg_ref, rb_ref,
             l2_ref, r2_ref, l1_ref, r1_ref, l0_ref, r0_ref,
             p0, p1, p2, p3, p4, p5, p6, p7, pb1_ref,
             pw2_ref, pb2_ref, pw3_ref, pb3_ref, pw4_ref, pb4_ref,
             pw5_ref, pb5_ref, pw6_ref, pb6_ref, y_ref):
  st = st_ref[...]
  l3 = _elu(_bn_apply(sl_ref[...], st, 0, lg_ref[...], lb_ref[...]))
  r3 = _elu(_bn_apply(sr_ref[...], st, 2, rg_ref[...], rb_ref[...]))
  parts = [l3, r3, l2_ref[...], r2_ref[...], l1_ref[...], r1_ref[...],
           l0_ref[...], r0_ref[...]]
  ws = [p0, p1, p2, p3, p4, p5, p6, p7]
  y = pb1_ref[...]
  for t, w in zip(parts, ws):
    y = y + jnp.dot(t, w[...], preferred_element_type=jnp.float32)
  y = _elu(y)
  y = _elu(jnp.dot(y, pw2_ref[...], preferred_element_type=jnp.float32)
           + pb2_ref[...])
  y = _elu(jnp.dot(y, pw3_ref[...], preferred_element_type=jnp.float32)
           + pb3_ref[...])
  y = _elu(jnp.dot(y, pw4_ref[...], preferred_element_type=jnp.float32)
           + pb4_ref[...])
  y = _elu(jnp.dot(y, pw5_ref[...], preferred_element_type=jnp.float32)
           + pb5_ref[...])
  y_ref[...] = (jnp.dot(y, pw6_ref[...], preferred_element_type=jnp.float32)
                + pb6_ref[...])


def _c2(sl, sr, st, lg, lb, rg, rb, jk, pws, pbs):
  blk = pl.BlockSpec((RB, DH), lambda i: (i, 0))
  stspec = pl.BlockSpec((8, DH), lambda i: (0, 0))
  vec = pl.BlockSpec((1, DH), lambda i: (0, 0))

  def wspec(a):
    return pl.BlockSpec(a.shape, lambda i: tuple(0 for _ in a.shape))

  p1s = pws[0]
  in_specs = ([blk, blk, stspec, vec, vec, vec, vec] + [blk] * 6
              + [wspec(w) for w in p1s] + [wspec(pbs[0])])
  args = [sl, sr, st, lg, lb, rg, rb] + list(jk) + list(p1s) + [pbs[0]]
  for w, b in zip(pws[1:], pbs[1:]):
    in_specs += [wspec(w), wspec(b)]
    args += [w, b]
  return pl.pallas_call(
      _c2_body,
      grid=(NRB,),
      in_specs=in_specs,
      out_specs=pl.BlockSpec((RB, 2), lambda i: (i, 0)),
      out_shape=jax.ShapeDtypeStruct((NN, 2), jnp.float32),
  )(*args)


def kernel(x, edge_index, W1, g1, b1, W2, g2, b2, Lll, Llr, Latt, Lg, Lb,
           Rll, Rlr, Ratt, Rg, Rb, pw1, pb1, pw2, pb2, pw3, pb3, pw4, pb4,
           pw5, pb5, pw6, pb6):
  f32 = jnp.float32
  src = edge_index[0].reshape(NC * NS, NBLK, BLK)
  dst = edge_index[1].reshape(NC * NS, NBLK, BLK)
  z72 = jnp.zeros((ZCH, DH + H), f32)
  z80 = jnp.zeros((ZCH, DH + 2 * H), f32)
  del f32

  r1 = lambda a: a.reshape(1, -1)
  h = _pre(x, W1, r1(g1), r1(b1), W2, r1(g2), r1(b2))
  hl, hr = h, h
  xll, xrl, xlr, xrr = _mm4(h, Lll[0], Llr[0], Rll[0], Rlr[0])

  ls, rs = [], []
  y = None
  for i in range(4):
    ndl = _edge_sum(xll, xrl, src, dst, Latt[i].reshape(-1), z72)[:, :NN, :]
    ndr = _edge_mean(xlr, xrr, src, dst, Ratt[i].reshape(-1), z80)[:, :NN, :]
    sl, sr, st = _b1(ndl, ndr, hl, hr)
    if i < 3:
      hl, hr, xll, xrl, xlr, xrr = _b2(
          sl, sr, st, r1(Lg[i]), r1(Lb[i]), r1(Rg[i]), r1(Rb[i]),
          Lll[i + 1], Llr[i + 1], Rll[i + 1], Rlr[i + 1])
      ls.append(hl)
      rs.append(hr)
    else:
      jk = [ls[2], rs[2], ls[1], rs[1], ls[0], rs[0]]
      p1s = [pw1[DH * k:DH * (k + 1)] for k in range(8)]
      pws = [p1s, pw2, pw3, pw4, pw5, pw6]
      pbs = [r1(pb1), r1(pb2), r1(pb3), r1(pb4), r1(pb5), r1(pb6)]
      y = _c2(sl, sr, st, r1(Lg[i]), r1(Lb[i]), r1(Rg[i]), r1(Rb[i]),
              jk, pws, pbs)
  return y


# speedup vs baseline: 1.0555x; 1.0555x over previous
import functools

import jax
import jax.numpy as jnp
from jax import lax
from jax.experimental import pallas as pl
from jax.experimental.pallas import tpu as pltpu
from jax.experimental.pallas import tpu_sc as plsc

NN = 10000
EE = 320000
H = 8
C = 8
DH = 64

NC = 2
NS = 16
LANES = 16

EDGES_PER_TILE = EE // (NC * NS)
BLK = 80
NBLK = EDGES_PER_TILE // BLK
NPAD = 10240
ROWS_PER_TILE = NPAD // NS
ZCH = 80

RB = 1000
NRB = NN // RB


NBUF = 3


def _make_edge_kernel(width, with_ones):
  mesh = plsc.VectorSubcoreMesh(core_axis_name="c", subcore_axis_name="s")

  @functools.partial(
      pl.kernel,
      out_type=jax.ShapeDtypeStruct((NC, NPAD, width), jnp.float32),
      mesh=mesh,
      scratch_types=[
          pltpu.VMEM_SHARED((NPAD, width), jnp.float32),
          pltpu.VMEM((NBLK, BLK), jnp.int32),
          pltpu.VMEM((NBLK, BLK), jnp.int32),
          [pltpu.VMEM((BLK, DH), jnp.float32) for _ in range(NBUF)],
          [pltpu.VMEM((BLK, DH), jnp.float32) for _ in range(NBUF)],
          [pltpu.VMEM((BLK, width), jnp.float32) for _ in range(NBUF)],
          [pltpu.VMEM((BLK,), jnp.int32) for _ in range(NBUF)],
          pltpu.VMEM((DH, LANES), jnp.float32),
          pltpu.SemaphoreType.DMA,
          [pltpu.SemaphoreType.DMA for _ in range(NBUF)],
          [pltpu.SemaphoreType.DMA for _ in range(NBUF)],
      ],
      compiler_params=pltpu.CompilerParams(
          needs_layout_passes=False, use_tc_tiling_on_sc=False),
  )
  def edge_kernel(xl_hbm, xr_hbm, src_hbm, dst_hbm, att_hbm, z_hbm,
                  acc_out, acc_sh, src_v, dst_v, xl_v, xr_v, prod_v,
                  sidx_v, att_v, asem, gsem, ssem):
    cid = lax.axis_index("c")
    sid = lax.axis_index("s")
    tid = cid * NS + sid

    pltpu.sync_copy(att_hbm, att_v)
    pltpu.sync_copy(src_hbm.at[tid], src_v)
    pltpu.sync_copy(dst_hbm.at[tid], dst_v)

    for j in range(ROWS_PER_TILE // ZCH):
      off = sid * ROWS_PER_TILE + j * ZCH
      pltpu.sync_copy(z_hbm, acc_sh.at[pl.ds(off, ZCH)])

    if with_ones:
      ones16 = jnp.ones((LANES,), jnp.float32)
      for s in range(NBUF):
        for g in range(BLK // LANES):
          rows = lax.iota(jnp.int32, LANES) + g * LANES
          for cc in range(DH + H, width):
            plsc.store_scatter(prod_v[s],
                               [rows, jnp.full((LANES,), cc, jnp.int32)],
                               ones16)

    plsc.subcore_barrier()

    def issue_gather(b, s):
      pltpu.async_copy(xl_hbm.at[src_v.at[b]], xl_v[s], gsem[s])
      pltpu.async_copy(xr_hbm.at[dst_v.at[b]], xr_v[s], gsem[s])

    def compute(b, s):
      @plsc.parallel_loop(0, (BLK // LANES) * H, unroll=4)
      def _(i, s=s):
        h = lax.bitwise_and(i, H - 1)
        g = lax.shift_right_logical(i, 3)
        rows = lax.iota(jnp.int32, LANES) + g * LANES
        colbase = h * C
        acc = jnp.zeros((LANES,), jnp.float32)
        xls = []
        for c in range(C):
          col = jnp.full((LANES,), c, jnp.int32) + colbase
          attv = att_v[colbase + c, :]
          xlv = plsc.load_gather(xl_v[s], [rows, col])
          xrv = plsc.load_gather(xr_v[s], [rows, col])
          sv = xlv + xrv
          sv = jnp.maximum(sv, 0.2 * sv)
          acc = acc + sv * attv
          xls.append((col, xlv))
        exh = jnp.exp(acc)
        plsc.store_scatter(prod_v[s],
                           [rows, jnp.full((LANES,), DH, jnp.int32) + h],
                           exh)
        for col, xlv in xls:
          plsc.store_scatter(prod_v[s], [rows, col], exh * xlv)
      for g in range(BLK // LANES):
        sidx_v[s][pl.ds(g * LANES, LANES)] = dst_v[b, pl.ds(g * LANES, LANES)]

    issue_gather(0, 0)
    issue_gather(1, 1)

    def step(b, s):
      sp = (s + 2) % NBUF

      @pl.when(b + 2 < NBLK)
      def _(b=b, sp=sp):
        issue_gather(b + 2, sp)

      pltpu.make_async_copy(xl_hbm.at[src_v.at[b]], xl_v[s], gsem[s]).wait()
      pltpu.make_async_copy(xr_hbm.at[dst_v.at[b]], xr_v[s], gsem[s]).wait()

      @pl.when(b >= NBUF)
      def _(s=s):
        pltpu.make_async_copy(prod_v[s], acc_sh.at[sidx_v[s]], ssem[s]).wait()

      compute(b, s)
      pltpu.async_copy(prod_v[s], acc_sh.at[sidx_v[s]], ssem[s], add=True)

    nfull = NBLK // NBUF

    def outer(o, carry):
      for s in range(NBUF):
        step(o * NBUF + s, s)
      return carry

    lax.fori_loop(0, nfull, outer, 0)
    for b in range(nfull * NBUF, NBLK):
      step(jnp.int32(b), b % NBUF)
    for s in range(NBUF):
      pltpu.make_async_copy(prod_v[s], acc_sh.at[sidx_v[s]], ssem[s]).wait()

    plsc.subcore_barrier()

    for j in range(ROWS_PER_TILE // ZCH):
      off = sid * ROWS_PER_TILE + j * ZCH
      pltpu.sync_copy(acc_sh.at[pl.ds(off, ZCH)],
                      acc_out.at[cid, pl.ds(off, ZCH)])

  return edge_kernel


_edge_sum = _make_edge_kernel(DH + H, False)
_edge_mean = _make_edge_kernel(DH + H + H, True)


def _elu(t):
  return jnp.where(t > 0, t, jnp.exp(jnp.minimum(t, 0.0)) - 1.0)


def _bn_full(t, g, b):
  m = jnp.mean(t, axis=0, keepdims=True)
  v = jnp.mean((t - m) * (t - m), axis=0, keepdims=True)
  return (t - m) / jnp.sqrt(v + 1e-5) * g + b


def _pre_body(x_ref, w1_ref, g1_ref, b1_ref, w2_ref, g2_ref, b2_ref, h_ref):
  t = jnp.dot(x_ref[...], w1_ref[...], preferred_element_type=jnp.float32)
  t = _elu(_bn_full(t, g1_ref[...], b1_ref[...]))
  t = jnp.dot(t, w2_ref[...], preferred_element_type=jnp.float32)
  h_ref[...] = _elu(_bn_full(t, g2_ref[...], b2_ref[...]))


def _pre(x, w1, g1, b1, w2, g2, b2):
  return pl.pallas_call(
      _pre_body,
      out_shape=jax.ShapeDtypeStruct((NN, DH), jnp.float32),
  )(x, w1, g1, b1, w2, g2, b2)


def _mm4_body(h_ref, wa_ref, wb_ref, wc_ref, wd_ref, oa, ob, oc, od):
  hv = h_ref[...]
  oa[...] = jnp.dot(hv, wa_ref[...], preferred_element_type=jnp.float32)
  ob[...] = jnp.dot(hv, wb_ref[...], preferred_element_type=jnp.float32)
  oc[...] = jnp.dot(hv, wc_ref[...], preferred_element_type=jnp.float32)
  od[...] = jnp.dot(hv, wd_ref[...], preferred_element_type=jnp.float32)


def _mm4(h, wa, wb, wc, wd):
  blk = pl.BlockSpec((RB, DH), lambda i: (i, 0))
  wspec = pl.BlockSpec((DH, DH), lambda i: (0, 0))
  return pl.pallas_call(
      _mm4_body,
      grid=(NRB,),
      in_specs=[blk, wspec, wspec, wspec, wspec],
      out_specs=[blk, blk, blk, blk],
      out_shape=[jax.ShapeDtypeStruct((NN, DH), jnp.float32)] * 4,
  )(h, wa, wb, wc, wd)


def _expand_mat():
  kk = lax.broadcasted_iota(jnp.int32, (H, DH), 1) // C
  hh = lax.broadcasted_iota(jnp.int32, (H, DH), 0)
  return (kk == hh).astype(jnp.float32)


def _b1_body(ndl_ref, ndr_ref, hl_ref, hr_ref, sl_ref, sr_ref, st_ref):
  i = pl.program_id(0)
  k = _expand_mat()
  ndl = ndl_ref[0] + ndl_ref[1]
  ndr = ndr_ref[0] + ndr_ref[1]
  den_l = jnp.dot(ndl[:, DH:DH + H], k, preferred_element_type=jnp.float32)
  gl = ndl[:, :DH] / (den_l + 1e-16)
  den_r = jnp.dot(ndr[:, DH:DH + H], k, preferred_element_type=jnp.float32)
  cnt = jnp.maximum(ndr[:, DH + H:DH + H + 1], 1.0)
  gr = ndr[:, :DH] / (den_r + 1e-16) / cnt
  sl = gl + hl_ref[...]
  sr = gr + hr_ref[...]
  sl_ref[...] = sl
  sr_ref[...] = sr
  z = jnp.zeros((1, DH), jnp.float32)
  st = jnp.concatenate(
      [jnp.sum(sl, axis=0, keepdims=True),
       jnp.sum(sl * sl, axis=0, keepdims=True),
       jnp.sum(sr, axis=0, keepdims=True),
       jnp.sum(sr * sr, axis=0, keepdims=True), z, z, z, z], axis=0)

  @pl.when(i == 0)
  def _():
    st_ref[...] = st

  @pl.when(i != 0)
  def _():
    st_ref[...] = st_ref[...] + st


def _b1(ndl, ndr, hl, hr):
  blk = pl.BlockSpec((RB, DH), lambda i: (i, 0))
  return pl.pallas_call(
      _b1_body,
      grid=(NRB,),
      in_specs=[
          pl.BlockSpec((NC, RB, DH + H), lambda i: (0, i, 0)),
          pl.BlockSpec((NC, RB, DH + 2 * H), lambda i: (0, i, 0)),
          blk, blk,
      ],
      out_specs=[blk, blk, pl.BlockSpec((8, DH), lambda i: (0, 0))],
      out_shape=[
          jax.ShapeDtypeStruct((NN, DH), jnp.float32),
          jax.ShapeDtypeStruct((NN, DH), jnp.float32),
          jax.ShapeDtypeStruct((8, DH), jnp.float32),
      ],
      compiler_params=pltpu.CompilerParams(
          dimension_semantics=("arbitrary",)),
  )(ndl, ndr, hl, hr)


def _bn_apply(s, st, row0, g, b):
  m = st[row0:row0 + 1] * (1.0 / NN)
  v = st[row0 + 1:row0 + 2] * (1.0 / NN) - m * m
  return (s - m) / jnp.sqrt(v + 1e-5) * g + b


def _b2_body(sl_ref, sr_ref, st_ref, lg_ref, lb_ref, rg_ref, rb_ref,
             wla_ref, wlb_ref, wra_ref, wrb_ref,
             hl_o, hr_o, xll_o, xrl_o, xlr_o, xrr_o):
  st = st_ref[...]
  hl = _elu(_bn_apply(sl_ref[...], st, 0, lg_ref[...], lb_ref[...]))
  hr = _elu(_bn_apply(sr_ref[...], st, 2, rg_ref[...], rb_ref[...]))
  hl_o[...] = hl
  hr_o[...] = hr
  xll_o[...] = jnp.dot(hl, wla_ref[...], preferred_element_type=jnp.float32)
  xrl_o[...] = jnp.dot(hl, wlb_ref[...], preferred_element_type=jnp.float32)
  xlr_o[...] = jnp.dot(hr, wra_ref[...], preferred_element_type=jnp.float32)
  xrr_o[...] = jnp.dot(hr, wrb_ref[...], preferred_element_type=jnp.float32)


def _b2(sl, sr, st, lg, lb, rg, rb, wla, wlb, wra, wrb):
  blk = pl.BlockSpec((RB, DH), lambda i: (i, 0))
  stspec = pl.BlockSpec((8, DH), lambda i: (0, 0))
  vec = pl.BlockSpec((1, DH), lambda i: (0, 0))
  wspec = pl.BlockSpec((DH, DH), lambda i: (0, 0))
  return pl.pallas_call(
      _b2_body,
      grid=(NRB,),
      in_specs=[blk, blk, stspec, vec, vec, vec, vec,
                wspec, wspec, wspec, wspec],
      out_specs=[blk] * 6,
      out_shape=[jax.ShapeDtypeStruct((NN, DH), jnp.float32)] * 6,
  )(sl, sr, st, lg, lb, rg, rb, wla, wlb, wra, wrb)


def _c2_body(sl_ref, sr_ref, st_ref, lg_ref, lb_ref, rg_ref, rb_ref,
             l2_ref, r2_ref, l1_ref, r1_ref, l0_ref, r0_ref,
             p0, p1, p2, p3, p4, p5, p6, p7, pb1_ref,
             pw2_ref, pb2_ref, pw3_ref, pb3_ref, pw4_ref, pb4_ref,
             pw5_ref, pb5_ref, pw6_ref, pb6_ref, y_ref):
  st = st_ref[...]
  l3 = _elu(_bn_apply(sl_ref[...], st, 0, lg_ref[...], lb_ref[...]))
  r3 = _elu(_bn_apply(sr_ref[...], st, 2, rg_ref[...], rb_ref[...]))
  parts = [l3, r3, l2_ref[...], r2_ref[...], l1_ref[...], r1_ref[...],
           l0_ref[...], r0_ref[...]]
  ws = [p0, p1, p2, p3, p4, p5, p6, p7]
  y = pb1_ref[...]
  for t, w in zip(parts, ws):
    y = y + jnp.dot(t, w[...], preferred_element_type=jnp.float32)
  y = _elu(y)
  y = _elu(jnp.dot(y, pw2_ref[...], preferred_element_type=jnp.float32)
           + pb2_ref[...])
  y = _elu(jnp.dot(y, pw3_ref[...], preferred_element_type=jnp.float32)
           + pb3_ref[...])
  y = _elu(jnp.dot(y, pw4_ref[...], preferred_element_type=jnp.float32)
           + pb4_ref[...])
  y = _elu(jnp.dot(y, pw5_ref[...], preferred_element_type=jnp.float32)
           + pb5_ref[...])
  y_ref[...] = (jnp.dot(y, pw6_ref[...], preferred_element_type=jnp.float32)
                + pb6_ref[...])


def _c2(sl, sr, st, lg, lb, rg, rb, jk, pws, pbs):
  blk = pl.BlockSpec((RB, DH), lambda i: (i, 0))
  stspec = pl.BlockSpec((8, DH), lambda i: (0, 0))
  vec = pl.BlockSpec((1, DH), lambda i: (0, 0))

  def wspec(a):
    return pl.BlockSpec(a.shape, lambda i: tuple(0 for _ in a.shape))

  p1s = pws[0]
  in_specs = ([blk, blk, stspec, vec, vec, vec, vec] + [blk] * 6
              + [wspec(w) for w in p1s] + [wspec(pbs[0])])
  args = [sl, sr, st, lg, lb, rg, rb] + list(jk) + list(p1s) + [pbs[0]]
  for w, b in zip(pws[1:], pbs[1:]):
    in_specs += [wspec(w), wspec(b)]
    args += [w, b]
  return pl.pallas_call(
      _c2_body,
      grid=(NRB,),
      in_specs=in_specs,
      out_specs=pl.BlockSpec((RB, 2), lambda i: (i, 0)),
      out_shape=jax.ShapeDtypeStruct((NN, 2), jnp.float32),
  )(*args)


def kernel(x, edge_index, W1, g1, b1, W2, g2, b2, Lll, Llr, Latt, Lg, Lb,
           Rll, Rlr, Ratt, Rg, Rb, pw1, pb1, pw2, pb2, pw3, pb3, pw4, pb4,
           pw5, pb5, pw6, pb6):
  f32 = jnp.float32
  src = edge_index[0].reshape(NC * NS, NBLK, BLK)
  dst = edge_index[1].reshape(NC * NS, NBLK, BLK)
  z72 = jnp.zeros((ZCH, DH + H), f32)
  z80 = jnp.zeros((ZCH, DH + 2 * H), f32)
  del f32

  r1 = lambda a: a.reshape(1, -1)
  h = _pre(x, W1, r1(g1), r1(b1), W2, r1(g2), r1(b2))
  hl, hr = h, h
  xll, xrl, xlr, xrr = _mm4(h, Lll[0], Llr[0], Rll[0], Rlr[0])

  ls, rs = [], []
  y = None
  for i in range(4):
    latt = jnp.broadcast_to(Latt[i].reshape(DH, 1), (DH, LANES))
    ratt = jnp.broadcast_to(Ratt[i].reshape(DH, 1), (DH, LANES))
    ndl = _edge_sum(xll, xrl, src, dst, latt, z72)[:, :NN, :]
    ndr = _edge_mean(xlr, xrr, src, dst, ratt, z80)[:, :NN, :]
    sl, sr, st = _b1(ndl, ndr, hl, hr)
    if i < 3:
      hl, hr, xll, xrl, xlr, xrr = _b2(
          sl, sr, st, r1(Lg[i]), r1(Lb[i]), r1(Rg[i]), r1(Rb[i]),
          Lll[i + 1], Llr[i + 1], Rll[i + 1], Rlr[i + 1])
      ls.append(hl)
      rs.append(hr)
    else:
      jk = [ls[2], rs[2], ls[1], rs[1], ls[0], rs[0]]
      p1s = [pw1[DH * k:DH * (k + 1)] for k in range(8)]
      pws = [p1s, pw2, pw3, pw4, pw5, pw6]
      pbs = [r1(pb1), r1(pb2), r1(pb3), r1(pb4), r1(pb5), r1(pb6)]
      y = _c2(sl, sr, st, r1(Lg[i]), r1(Lb[i]), r1(Rg[i]), r1(Rb[i]),
              jk, pws, pbs)
  return y


# speedup vs baseline: 1.1899x; 1.1274x over previous
import functools

import jax
import jax.numpy as jnp
from jax import lax
from jax.experimental import pallas as pl
from jax.experimental.pallas import tpu as pltpu
from jax.experimental.pallas import tpu_sc as plsc

NN = 10000
EE = 320000
H = 8
C = 8
DH = 64

NC = 2
NS = 16
LANES = 16

EDGES_PER_TILE = EE // (NC * NS)
BLK = 80
NBLK = EDGES_PER_TILE // BLK
NPAD = 10240
ROWS_PER_TILE = NPAD // NS
ZCH = 80

RB = 1000
NRB = NN // RB


NBUF = 3


def _make_edge_kernel(width, with_ones):
  mesh = plsc.VectorSubcoreMesh(core_axis_name="c", subcore_axis_name="s")

  @functools.partial(
      pl.kernel,
      out_type=jax.ShapeDtypeStruct((NC, NPAD, width), jnp.float32),
      mesh=mesh,
      scratch_types=[
          pltpu.VMEM_SHARED((NPAD, width), jnp.float32),
          pltpu.VMEM((NBLK, BLK), jnp.int32),
          pltpu.VMEM((NBLK, BLK), jnp.int32),
          [pltpu.VMEM((BLK, DH), jnp.float32) for _ in range(NBUF)],
          [pltpu.VMEM((BLK, DH), jnp.float32) for _ in range(NBUF)],
          [pltpu.VMEM((BLK, width), jnp.float32) for _ in range(NBUF)],
          [pltpu.VMEM((BLK,), jnp.int32) for _ in range(NBUF)],
          pltpu.VMEM((DH, LANES), jnp.float32),
          pltpu.SemaphoreType.DMA,
          [pltpu.SemaphoreType.DMA for _ in range(NBUF)],
          [pltpu.SemaphoreType.DMA for _ in range(NBUF)],
      ],
      compiler_params=pltpu.CompilerParams(
          needs_layout_passes=False, use_tc_tiling_on_sc=False),
  )
  def edge_kernel(xl_hbm, xr_hbm, src_hbm, dst_hbm, att_hbm, z_hbm,
                  acc_out, acc_sh, src_v, dst_v, xl_v, xr_v, prod_v,
                  sidx_v, att_v, asem, gsem, ssem):
    cid = lax.axis_index("c")
    sid = lax.axis_index("s")
    tid = cid * NS + sid

    pltpu.sync_copy(att_hbm, att_v)
    pltpu.sync_copy(src_hbm.at[tid], src_v)
    pltpu.sync_copy(dst_hbm.at[tid], dst_v)

    for j in range(ROWS_PER_TILE // ZCH):
      off = sid * ROWS_PER_TILE + j * ZCH
      pltpu.sync_copy(z_hbm, acc_sh.at[pl.ds(off, ZCH)])

    if with_ones:
      ones16 = jnp.ones((LANES,), jnp.float32)
      for s in range(NBUF):
        for g in range(BLK // LANES):
          rows = lax.iota(jnp.int32, LANES) + g * LANES
          for cc in range(DH + H, width):
            plsc.store_scatter(prod_v[s],
                               [rows, jnp.full((LANES,), cc, jnp.int32)],
                               ones16)

    plsc.subcore_barrier()

    def issue_gather(b, s):
      pltpu.async_copy(xl_hbm.at[src_v.at[b]], xl_v[s], gsem[s])
      pltpu.async_copy(xr_hbm.at[dst_v.at[b]], xr_v[s], gsem[s])

    def compute(b, s):
      @plsc.parallel_loop(0, (BLK // LANES) * H, unroll=8)
      def _(i, s=s):
        h = lax.bitwise_and(i, H - 1)
        g = lax.shift_right_logical(i, 3)
        rows = lax.iota(jnp.int32, LANES) + g * LANES
        colbase = h * C
        acc = jnp.zeros((LANES,), jnp.float32)
        xls = []
        for c in range(C):
          col = jnp.full((LANES,), c, jnp.int32) + colbase
          attv = att_v[colbase + c, :]
          xlv = plsc.load_gather(xl_v[s], [rows, col])
          xrv = plsc.load_gather(xr_v[s], [rows, col])
          sv = xlv + xrv
          sv = jnp.maximum(sv, 0.2 * sv)
          acc = acc + sv * attv
          xls.append((col, xlv))
        exh = jnp.exp(acc)
        plsc.store_scatter(prod_v[s],
                           [rows, jnp.full((LANES,), DH, jnp.int32) + h],
                           exh)
        for col, xlv in xls:
          plsc.store_scatter(prod_v[s], [rows, col], exh * xlv)
      for g in range(BLK // LANES):
        sidx_v[s][pl.ds(g * LANES, LANES)] = dst_v[b, pl.ds(g * LANES, LANES)]

    issue_gather(0, 0)
    issue_gather(1, 1)

    def step(b, s):
      sp = (s + 2) % NBUF

      @pl.when(b + 2 < NBLK)
      def _(b=b, sp=sp):
        issue_gather(b + 2, sp)

      pltpu.make_async_copy(xl_hbm.at[src_v.at[b]], xl_v[s], gsem[s]).wait()
      pltpu.make_async_copy(xr_hbm.at[dst_v.at[b]], xr_v[s], gsem[s]).wait()

      @pl.when(b >= NBUF)
      def _(s=s):
        pltpu.make_async_copy(prod_v[s], acc_sh.at[sidx_v[s]], ssem[s]).wait()

      compute(b, s)
      pltpu.async_copy(prod_v[s], acc_sh.at[sidx_v[s]], ssem[s], add=True)

    nfull = NBLK // NBUF

    def outer(o, carry):
      for s in range(NBUF):
        step(o * NBUF + s, s)
      return carry

    lax.fori_loop(0, nfull, outer, 0)
    for b in range(nfull * NBUF, NBLK):
      step(jnp.int32(b), b % NBUF)
    for s in range(NBUF):
      pltpu.make_async_copy(prod_v[s], acc_sh.at[sidx_v[s]], ssem[s]).wait()

    plsc.subcore_barrier()

    for j in range(ROWS_PER_TILE // ZCH):
      off = sid * ROWS_PER_TILE + j * ZCH
      pltpu.sync_copy(acc_sh.at[pl.ds(off, ZCH)],
                      acc_out.at[cid, pl.ds(off, ZCH)])

  return edge_kernel


_edge_sum = _make_edge_kernel(DH + H, False)
_edge_mean = _make_edge_kernel(DH + H + H, True)


def _elu(t):
  return jnp.where(t > 0, t, jnp.exp(jnp.minimum(t, 0.0)) - 1.0)


def _bn_full(t, g, b):
  m = jnp.mean(t, axis=0, keepdims=True)
  v = jnp.mean((t - m) * (t - m), axis=0, keepdims=True)
  return (t - m) / jnp.sqrt(v + 1e-5) * g + b


def _pre_body(x_ref, w1_ref, g1_ref, b1_ref, w2_ref, g2_ref, b2_ref, h_ref):
  t = jnp.dot(x_ref[...], w1_ref[...], preferred_element_type=jnp.float32)
  t = _elu(_bn_full(t, g1_ref[...], b1_ref[...]))
  t = jnp.dot(t, w2_ref[...], preferred_element_type=jnp.float32)
  h_ref[...] = _elu(_bn_full(t, g2_ref[...], b2_ref[...]))


def _pre(x, w1, g1, b1, w2, g2, b2):
  return pl.pallas_call(
      _pre_body,
      out_shape=jax.ShapeDtypeStruct((NN, DH), jnp.float32),
  )(x, w1, g1, b1, w2, g2, b2)


def _mm4_body(h_ref, wa_ref, wb_ref, wc_ref, wd_ref, oa, ob, oc, od):
  hv = h_ref[...]
  oa[...] = jnp.dot(hv, wa_ref[...], preferred_element_type=jnp.float32)
  ob[...] = jnp.dot(hv, wb_ref[...], preferred_element_type=jnp.float32)
  oc[...] = jnp.dot(hv, wc_ref[...], preferred_element_type=jnp.float32)
  od[...] = jnp.dot(hv, wd_ref[...], preferred_element_type=jnp.float32)


def _mm4(h, wa, wb, wc, wd):
  blk = pl.BlockSpec((RB, DH), lambda i: (i, 0))
  wspec = pl.BlockSpec((DH, DH), lambda i: (0, 0))
  return pl.pallas_call(
      _mm4_body,
      grid=(NRB,),
      in_specs=[blk, wspec, wspec, wspec, wspec],
      out_specs=[blk, blk, blk, blk],
      out_shape=[jax.ShapeDtypeStruct((NN, DH), jnp.float32)] * 4,
  )(h, wa, wb, wc, wd)


def _expand_mat():
  kk = lax.broadcasted_iota(jnp.int32, (H, DH), 1) // C
  hh = lax.broadcasted_iota(jnp.int32, (H, DH), 0)
  return (kk == hh).astype(jnp.float32)


def _b1_body(ndl_ref, ndr_ref, hl_ref, hr_ref, sl_ref, sr_ref, st_ref):
  i = pl.program_id(0)
  k = _expand_mat()
  ndl = ndl_ref[0] + ndl_ref[1]
  ndr = ndr_ref[0] + ndr_ref[1]
  den_l = jnp.dot(ndl[:, DH:DH + H], k, preferred_element_type=jnp.float32)
  gl = ndl[:, :DH] / (den_l + 1e-16)
  den_r = jnp.dot(ndr[:, DH:DH + H], k, preferred_element_type=jnp.float32)
  cnt = jnp.maximum(ndr[:, DH + H:DH + H + 1], 1.0)
  gr = ndr[:, :DH] / (den_r + 1e-16) / cnt
  sl = gl + hl_ref[...]
  sr = gr + hr_ref[...]
  sl_ref[...] = sl
  sr_ref[...] = sr
  z = jnp.zeros((1, DH), jnp.float32)
  st = jnp.concatenate(
      [jnp.sum(sl, axis=0, keepdims=True),
       jnp.sum(sl * sl, axis=0, keepdims=True),
       jnp.sum(sr, axis=0, keepdims=True),
       jnp.sum(sr * sr, axis=0, keepdims=True), z, z, z, z], axis=0)

  @pl.when(i == 0)
  def _():
    st_ref[...] = st

  @pl.when(i != 0)
  def _():
    st_ref[...] = st_ref[...] + st


def _b1(ndl, ndr, hl, hr):
  blk = pl.BlockSpec((RB, DH), lambda i: (i, 0))
  return pl.pallas_call(
      _b1_body,
      grid=(NRB,),
      in_specs=[
          pl.BlockSpec((NC, RB, DH + H), lambda i: (0, i, 0)),
          pl.BlockSpec((NC, RB, DH + 2 * H), lambda i: (0, i, 0)),
          blk, blk,
      ],
      out_specs=[blk, blk, pl.BlockSpec((8, DH), lambda i: (0, 0))],
      out_shape=[
          jax.ShapeDtypeStruct((NN, DH), jnp.float32),
          jax.ShapeDtypeStruct((NN, DH), jnp.float32),
          jax.ShapeDtypeStruct((8, DH), jnp.float32),
      ],
      compiler_params=pltpu.CompilerParams(
          dimension_semantics=("arbitrary",)),
  )(ndl, ndr, hl, hr)


def _bn_apply(s, st, row0, g, b):
  m = st[row0:row0 + 1] * (1.0 / NN)
  v = st[row0 + 1:row0 + 2] * (1.0 / NN) - m * m
  return (s - m) / jnp.sqrt(v + 1e-5) * g + b


def _b2_body(sl_ref, sr_ref, st_ref, lg_ref, lb_ref, rg_ref, rb_ref,
             wla_ref, wlb_ref, wra_ref, wrb_ref,
             hl_o, hr_o, xll_o, xrl_o, xlr_o, xrr_o):
  st = st_ref[...]
  hl = _elu(_bn_apply(sl_ref[...], st, 0, lg_ref[...], lb_ref[...]))
  hr = _elu(_bn_apply(sr_ref[...], st, 2, rg_ref[...], rb_ref[...]))
  hl_o[...] = hl
  hr_o[...] = hr
  xll_o[...] = jnp.dot(hl, wla_ref[...], preferred_element_type=jnp.float32)
  xrl_o[...] = jnp.dot(hl, wlb_ref[...], preferred_element_type=jnp.float32)
  xlr_o[...] = jnp.dot(hr, wra_ref[...], preferred_element_type=jnp.float32)
  xrr_o[...] = jnp.dot(hr, wrb_ref[...], preferred_element_type=jnp.float32)


def _b2(sl, sr, st, lg, lb, rg, rb, wla, wlb, wra, wrb):
  blk = pl.BlockSpec((RB, DH), lambda i: (i, 0))
  stspec = pl.BlockSpec((8, DH), lambda i: (0, 0))
  vec = pl.BlockSpec((1, DH), lambda i: (0, 0))
  wspec = pl.BlockSpec((DH, DH), lambda i: (0, 0))
  return pl.pallas_call(
      _b2_body,
      grid=(NRB,),
      in_specs=[blk, blk, stspec, vec, vec, vec, vec,
                wspec, wspec, wspec, wspec],
      out_specs=[blk] * 6,
      out_shape=[jax.ShapeDtypeStruct((NN, DH), jnp.float32)] * 6,
  )(sl, sr, st, lg, lb, rg, rb, wla, wlb, wra, wrb)


def _c2_body(sl_ref, sr_ref, st_ref, lg_ref, lb_ref, rg_ref, rb_ref,
             l2_ref, r2_ref, l1_ref, r1_ref, l0_ref, r0_ref,
             p0, p1, p2, p3, p4, p5, p6, p7, pb1_ref,
             pw2_ref, pb2_ref, pw3_ref, pb3_ref, pw4_ref, pb4_ref,
             pw5_ref, pb5_ref, pw6_ref, pb6_ref, y_ref):
  st = st_ref[...]
  l3 = _elu(_bn_apply(sl_ref[...], st, 0, lg_ref[...], lb_ref[...]))
  r3 = _elu(_bn_apply(sr_ref[...], st, 2, rg_ref[...], rb_ref[...]))
  parts = [l3, r3, l2_ref[...], r2_ref[...], l1_ref[...], r1_ref[...],
           l0_ref[...], r0_ref[...]]
  ws = [p0, p1, p2, p3, p4, p5, p6, p7]
  y = pb1_ref[...]
  for t, w in zip(parts, ws):
    y = y + jnp.dot(t, w[...], preferred_element_type=jnp.float32)
  y = _elu(y)
  y = _elu(jnp.dot(y, pw2_ref[...], preferred_element_type=jnp.float32)
           + pb2_ref[...])
  y = _elu(jnp.dot(y, pw3_ref[...], preferred_element_type=jnp.float32)
           + pb3_ref[...])
  y = _elu(jnp.dot(y, pw4_ref[...], preferred_element_type=jnp.float32)
           + pb4_ref[...])
  y = _elu(jnp.dot(y, pw5_ref[...], preferred_element_type=jnp.float32)
           + pb5_ref[...])
  y_ref[...] = (jnp.dot(y, pw6_ref[...], preferred_element_type=jnp.float32)
                + pb6_ref[...])


def _c2(sl, sr, st, lg, lb, rg, rb, jk, pws, pbs):
  blk = pl.BlockSpec((RB, DH), lambda i: (i, 0))
  stspec = pl.BlockSpec((8, DH), lambda i: (0, 0))
  vec = pl.BlockSpec((1, DH), lambda i: (0, 0))

  def wspec(a):
    return pl.BlockSpec(a.shape, lambda i: tuple(0 for _ in a.shape))

  p1s = pws[0]
  in_specs = ([blk, blk, stspec, vec, vec, vec, vec] + [blk] * 6
              + [wspec(w) for w in p1s] + [wspec(pbs[0])])
  args = [sl, sr, st, lg, lb, rg, rb] + list(jk) + list(p1s) + [pbs[0]]
  for w, b in zip(pws[1:], pbs[1:]):
    in_specs += [wspec(w), wspec(b)]
    args += [w, b]
  return pl.pallas_call(
      _c2_body,
      grid=(NRB,),
      in_specs=in_specs,
      out_specs=pl.BlockSpec((RB, 2), lambda i: (i, 0)),
      out_shape=jax.ShapeDtypeStruct((NN, 2), jnp.float32),
  )(*args)


def kernel(x, edge_index, W1, g1, b1, W2, g2, b2, Lll, Llr, Latt, Lg, Lb,
           Rll, Rlr, Ratt, Rg, Rb, pw1, pb1, pw2, pb2, pw3, pb3, pw4, pb4,
           pw5, pb5, pw6, pb6):
  f32 = jnp.float32
  src = edge_index[0].reshape(NC * NS, NBLK, BLK)
  dst = edge_index[1].reshape(NC * NS, NBLK, BLK)
  z72 = jnp.zeros((ZCH, DH + H), f32)
  z80 = jnp.zeros((ZCH, DH + 2 * H), f32)
  del f32

  r1 = lambda a: a.reshape(1, -1)
  h = _pre(x, W1, r1(g1), r1(b1), W2, r1(g2), r1(b2))
  hl, hr = h, h
  xll, xrl, xlr, xrr = _mm4(h, Lll[0], Llr[0], Rll[0], Rlr[0])

  ls, rs = [], []
  y = None
  for i in range(4):
    latt = jnp.broadcast_to(Latt[i].reshape(DH, 1), (DH, LANES))
    ratt = jnp.broadcast_to(Ratt[i].reshape(DH, 1), (DH, LANES))
    ndl = _edge_sum(xll, xrl, src, dst, latt, z72)[:, :NN, :]
    ndr = _edge_mean(xlr, xrr, src, dst, ratt, z80)[:, :NN, :]
    sl, sr, st = _b1(ndl, ndr, hl, hr)
    if i < 3:
      hl, hr, xll, xrl, xlr, xrr = _b2(
          sl, sr, st, r1(Lg[i]), r1(Lb[i]), r1(Rg[i]), r1(Rb[i]),
          Lll[i + 1], Llr[i + 1], Rll[i + 1], Rlr[i + 1])
      ls.append(hl)
      rs.append(hr)
    else:
      jk = [ls[2], rs[2], ls[1], rs[1], ls[0], rs[0]]
      p1s = [pw1[DH * k:DH * (k + 1)] for k in range(8)]
      pws = [p1s, pw2, pw3, pw4, pw5, pw6]
      pbs = [r1(pb1), r1(pb2), r1(pb3), r1(pb4), r1(pb5), r1(pb6)]
      y = _c2(sl, sr, st, r1(Lg[i]), r1(Lb[i]), r1(Rg[i]), r1(Rb[i]),
              jk, pws, pbs)
  return y
